# Initial kernel scaffold; baseline (speedup 1.0000x reference)
import jax
import jax.numpy as jnp
from jax.experimental import pallas as pl


def kernel(h, x, params, edge_index, n_node):
    raise NotImplementedError("write your pallas kernel here")


# speedup vs baseline: 1.1533x; 1.1533x over previous
import jax
import jax.numpy as jnp
from jax.experimental import pallas as pl

N = 50000
E = 800000
H = 64
NRBF = 50
NG = 10


def _readout_body(h_ref, w1_ref, b1_ref, w2_ref, b2_ref, ms_ref, o_ref):
    g = pl.program_id(0)
    y = jnp.sum(h_ref[...], axis=0, keepdims=True)
    t = y @ w1_ref[...] + b1_ref[...]
    t = t * jax.nn.sigmoid(t)
    out = t @ w2_ref[...] + b2_ref[...]
    o_ref[pl.ds(g, 1), :] = ms_ref[0, 1] * out + ms_ref[0, 0]


def _readout(h, W1, b1, W2, b2, mean, std):
    per_g = h.shape[0] // NG
    ms = jnp.stack([mean, std]).reshape(1, 2)
    return pl.pallas_call(
        _readout_body,
        grid=(NG,),
        in_specs=[
            pl.BlockSpec((per_g, H), lambda g: (g, 0)),
            pl.BlockSpec((H, H), lambda g: (0, 0)),
            pl.BlockSpec((1, H), lambda g: (0, 0)),
            pl.BlockSpec((H, 1), lambda g: (0, 0)),
            pl.BlockSpec((1, 1), lambda g: (0, 0)),
            pl.BlockSpec((1, 2), lambda g: (0, 0)),
        ],
        out_specs=pl.BlockSpec((NG, 1), lambda g: (0, 0)),
        out_shape=jax.ShapeDtypeStruct((NG, 1), jnp.float32),
    )(h, W1, b1.reshape(1, H), W2, b2.reshape(1, 1), ms)


def kernel(h, x, params, edge_index, n_node):
    src = edge_index[0]
    dst = edge_index[1]
    mu = jnp.linspace(0.0, 8.0, NRBF)
    gamma = 10.0

    dx = jnp.take(x, src, axis=0) - jnp.take(x, dst, axis=0)
    d = jnp.sqrt(jnp.sum(dx * dx, axis=-1) + 1e-8)
    rbf = jnp.exp(-gamma * (d[:, None] - mu[None, :]) ** 2)

    h = jax.nn.silu(h @ params['W_in'] + params['b_in'])
    for lp in params['layers']:
        We = lp['W_e']
        p_src = h @ We[:H]
        p_dst = h @ We[H:2 * H]
        rbfp = rbf @ We[2 * H:]
        mpre = (jnp.take(p_src, src, axis=0) + jnp.take(p_dst, dst, axis=0)
                + rbfp + lp['b_e'])
        m = jax.nn.silu(mpre)
        a = jax.nn.sigmoid(m @ lp['w_a'])
        agg = jax.ops.segment_sum(m * a[:, None], dst, num_segments=h.shape[0])
        Wu = lp['W_u']
        h = h + jax.nn.silu(h @ Wu[:H] + agg @ Wu[H:] + lp['b_u'])

    return _readout(h, params['W1'], params['b1'], params['W2'], params['b2'],
                    params['mean'], params['std'])


# speedup vs baseline: 1.6675x; 1.4459x over previous
import functools

import jax
import jax.numpy as jnp
from jax import lax
from jax.experimental import pallas as pl
from jax.experimental.pallas import tpu as pltpu
from jax.experimental.pallas import tpu_sc as plsc

N = 50000
E = 800000
H = 64
NRBF = 50
NG = 10

NW = 32
CHUNK = 640
IROWS = CHUNK // 128
NCHUNK = E // CHUNK
HALF = N // 2
TPC = 16
SCHUNK = 256
SIROWS = SCHUNK // 128
NSCHUNK = E // SCHUNK
ACC_PER_TILE = 1664
ACC_ROWS = TPC * ACC_PER_TILE
DUMMY_SPREAD = 1024
COPY_CH = 200
NCOPY = HALF // COPY_CH

_mesh = plsc.VectorSubcoreMesh(core_axis_name="c", subcore_axis_name="s",
                               num_cores=2, num_subcores=16)
_sc_params = pltpu.CompilerParams(use_tc_tiling_on_sc=False)


def _make_gather2(width):
    def body(tab_s, tab_d, src_hbm, dst_hbm, out_s, out_d, idx_v, rows_v, sem):
        wid = lax.axis_index("s") * 2 + lax.axis_index("c")

        @pl.loop(0, (NCHUNK + NW - 1) // NW)
        def _(i):
            c = wid + i * NW

            @pl.when(c < NCHUNK)
            def _():
                base = c * CHUNK
                pltpu.sync_copy(src_hbm.at[pl.ds(base, CHUNK)], idx_v)
                hs = [pltpu.async_copy(tab_s.at[idx_v.at[pl.ds(r * 128, 128)]],
                                       rows_v.at[pl.ds(r * 128, 128)], sem)
                      for r in range(IROWS)]
                for hh in hs:
                    hh.wait()
                pltpu.sync_copy(rows_v, out_s.at[pl.ds(base, CHUNK)])

                pltpu.sync_copy(dst_hbm.at[pl.ds(base, CHUNK)], idx_v)
                hs = [pltpu.async_copy(tab_d.at[idx_v.at[pl.ds(r * 128, 128)]],
                                       rows_v.at[pl.ds(r * 128, 128)], sem)
                      for r in range(IROWS)]
                for hh in hs:
                    hh.wait()
                pltpu.sync_copy(rows_v, out_d.at[pl.ds(base, CHUNK)])

    shp = jax.ShapeDtypeStruct((E, width), jnp.float32)
    return pl.kernel(
        body, mesh=_mesh, compiler_params=_sc_params,
        out_type=[shp, shp],
        scratch_types=[
            pltpu.VMEM((CHUNK,), jnp.int32),
            pltpu.VMEM((CHUNK, width), jnp.float32),
            pltpu.SemaphoreType.DMA,
        ])


_gather_h = _make_gather2(H)
_gather_x = _make_gather2(16)


def _scatter_body(contrib, li0, li1, zeros_hbm, out_hbm, idx_v, rows_v, acc, sem):
    core = lax.axis_index("c")
    s = lax.axis_index("s")

    pltpu.sync_copy(zeros_hbm, rows_v)
    for r in range(6):
        pltpu.sync_copy(rows_v,
                        acc.at[pl.ds(s * ACC_PER_TILE + r * SCHUNK, SCHUNK)])
    pltpu.sync_copy(rows_v.at[pl.ds(0, 128)],
                    acc.at[pl.ds(s * ACC_PER_TILE + 6 * SCHUNK, 128)])
    plsc.subcore_barrier()

    def edge_loop(li_hbm):
        @pl.loop(0, (NSCHUNK + TPC - 1) // TPC)
        def _(i):
            c = s + i * TPC

            @pl.when(c < NSCHUNK)
            def _():
                pltpu.sync_copy(contrib.at[pl.ds(c * SCHUNK, SCHUNK)], rows_v)
                pltpu.sync_copy(li_hbm.at[c], idx_v)
                for r in range(SIROWS):
                    pltpu.sync_copy(rows_v.at[pl.ds(r * 128, 128)],
                                    acc.at[idx_v.at[r]], add=True)

    @pl.when(core == 0)
    def _():
        edge_loop(li0)

    @pl.when(core == 1)
    def _():
        edge_loop(li1)

    plsc.subcore_barrier()

    @pl.loop(0, (NCOPY + TPC - 1) // TPC)
    def _(i):
        q = s + i * TPC

        @pl.when(q < NCOPY)
        def _():
            pltpu.sync_copy(acc.at[pl.ds(q * COPY_CH, COPY_CH)],
                            rows_v.at[pl.ds(0, COPY_CH)])
            pltpu.sync_copy(rows_v.at[pl.ds(0, COPY_CH)],
                            out_hbm.at[pl.ds(core * HALF + q * COPY_CH,
                                             COPY_CH)])


_scatter = pl.kernel(
    _scatter_body, mesh=_mesh, compiler_params=_sc_params,
    out_type=jax.ShapeDtypeStruct((N, H), jnp.float32),
    scratch_types=[
        pltpu.VMEM((SIROWS, 128), jnp.int32),
        pltpu.VMEM((SCHUNK, H), jnp.float32),
        pltpu.VMEM_SHARED((ACC_ROWS, H), jnp.float32),
        pltpu.SemaphoreType.DMA,
    ])


BN = 1000


def _mm(a, b):
    return jnp.dot(a, b, preferred_element_type=jnp.float32,
                   precision=jax.lax.Precision.HIGHEST)


def _prep_nodes_body(h_ref, win_ref, bin_ref, wes_ref, wed_ref,
                     h0_ref, ts_ref, td_ref):
    t = _mm(h_ref[...], win_ref[...]) + bin_ref[...]
    h0 = t * jax.nn.sigmoid(t)
    h0_ref[...] = h0
    ts_ref[...] = _mm(h0, wes_ref[...])
    td_ref[...] = _mm(h0, wed_ref[...])


def _prep_nodes(h, W_in, b_in, wes, wed):
    return pl.pallas_call(
        _prep_nodes_body,
        grid=(N // BN,),
        in_specs=[
            pl.BlockSpec((BN, H), lambda i: (i, 0)),
            pl.BlockSpec((H, H), lambda i: (0, 0)),
            pl.BlockSpec((1, H), lambda i: (0, 0)),
            pl.BlockSpec((H, H), lambda i: (0, 0)),
            pl.BlockSpec((H, H), lambda i: (0, 0)),
        ],
        out_specs=[pl.BlockSpec((BN, H), lambda i: (i, 0))] * 3,
        out_shape=[jax.ShapeDtypeStruct((N, H), jnp.float32)] * 3,
    )(h, W_in, b_in.reshape(1, H), wes, wed)


def _prep_li_body(dst_ref, li0_ref, li1_ref):
    d = dst_ref[...]
    cid = lax.broadcasted_iota(jnp.int32, d.shape, 0)
    mid = lax.broadcasted_iota(jnp.int32, d.shape, 1)
    lane = lax.broadcasted_iota(jnp.int32, d.shape, 2)
    dummy = HALF + ((cid * SCHUNK + mid * 128 + lane) % DUMMY_SPREAD)
    li0_ref[...] = jnp.where(d < HALF, d, dummy)
    li1_ref[...] = jnp.where(d >= HALF, d - HALF, dummy)


def _prep_li(dst3d):
    return pl.pallas_call(
        _prep_li_body,
        grid=(1,),
        in_specs=[pl.BlockSpec((NSCHUNK, SIROWS, 128), lambda i: (0, 0, 0))],
        out_specs=[pl.BlockSpec((NSCHUNK, SIROWS, 128),
                                lambda i: (0, 0, 0))] * 2,
        out_shape=[jax.ShapeDtypeStruct((NSCHUNK, SIROWS, 128),
                                        jnp.int32)] * 2,
    )(dst3d)


def _prep_d_body(xs_ref, xd_ref, d_ref):
    dx = xs_ref[...] - xd_ref[...]
    d_ref[...] = jnp.sqrt(jnp.sum(dx * dx, axis=1, keepdims=True) + 1e-8)


def _prep_d(xs, xd):
    return pl.pallas_call(
        _prep_d_body,
        grid=(NCHUNK,),
        in_specs=[pl.BlockSpec((CHUNK, 16), lambda i: (i, 0))] * 2,
        out_specs=pl.BlockSpec((CHUNK, 1), lambda i: (i, 0)),
        out_shape=jax.ShapeDtypeStruct((E, 1), jnp.float32),
    )(xs, xd)


def _edge_body(gs_ref, gd_ref, d_ref, mu_ref, wer_ref, be_ref, wa_ref, o_ref):
    d = d_ref[...]
    rbf = jnp.exp(-10.0 * (d - mu_ref[...]) ** 2)
    mpre = (gs_ref[...] + gd_ref[...]
            + _mm(rbf, wer_ref[...])
            + be_ref[...])
    m = mpre * jax.nn.sigmoid(mpre)
    a = jax.nn.sigmoid(_mm(m, wa_ref[...]))
    o_ref[...] = m * a


def _edge(gs, gd, d, mu_pad, wer_pad, b_e, w_a):
    return pl.pallas_call(
        _edge_body,
        grid=(NCHUNK,),
        in_specs=[
            pl.BlockSpec((CHUNK, H), lambda i: (i, 0)),
            pl.BlockSpec((CHUNK, H), lambda i: (i, 0)),
            pl.BlockSpec((CHUNK, 1), lambda i: (i, 0)),
            pl.BlockSpec((1, H), lambda i: (0, 0)),
            pl.BlockSpec((H, H), lambda i: (0, 0)),
            pl.BlockSpec((1, H), lambda i: (0, 0)),
            pl.BlockSpec((H, 1), lambda i: (0, 0)),
        ],
        out_specs=pl.BlockSpec((CHUNK, H), lambda i: (i, 0)),
        out_shape=jax.ShapeDtypeStruct((E, H), jnp.float32),
    )(gs, gd, d, mu_pad, wer_pad, b_e.reshape(1, H), w_a.reshape(H, 1))


def _node_body(h_ref, agg_ref, wuh_ref, wua_ref, bu_ref, wes_ref, wed_ref,
               h_out, ts_out, td_out):
    t = (_mm(h_ref[...], wuh_ref[...]) + _mm(agg_ref[...], wua_ref[...])
         + bu_ref[...])
    hn = h_ref[...] + t * jax.nn.sigmoid(t)
    h_out[...] = hn
    ts_out[...] = _mm(hn, wes_ref[...])
    td_out[...] = _mm(hn, wed_ref[...])


def _node(h, agg, wuh, wua, b_u, wes, wed):
    return pl.pallas_call(
        _node_body,
        grid=(N // BN,),
        in_specs=[
            pl.BlockSpec((BN, H), lambda i: (i, 0)),
            pl.BlockSpec((BN, H), lambda i: (i, 0)),
            pl.BlockSpec((H, H), lambda i: (0, 0)),
            pl.BlockSpec((H, H), lambda i: (0, 0)),
            pl.BlockSpec((1, H), lambda i: (0, 0)),
            pl.BlockSpec((H, H), lambda i: (0, 0)),
            pl.BlockSpec((H, H), lambda i: (0, 0)),
        ],
        out_specs=[pl.BlockSpec((BN, H), lambda i: (i, 0))] * 3,
        out_shape=[jax.ShapeDtypeStruct((N, H), jnp.float32)] * 3,
    )(h, agg, wuh, wua, b_u.reshape(1, H), wes, wed)


def _node_last_body(h_ref, agg_ref, wuh_ref, wua_ref, bu_ref, h_out):
    t = (_mm(h_ref[...], wuh_ref[...]) + _mm(agg_ref[...], wua_ref[...])
         + bu_ref[...])
    h_out[...] = h_ref[...] + t * jax.nn.sigmoid(t)


def _node_last(h, agg, wuh, wua, b_u):
    return pl.pallas_call(
        _node_last_body,
        grid=(N // BN,),
        in_specs=[
            pl.BlockSpec((BN, H), lambda i: (i, 0)),
            pl.BlockSpec((BN, H), lambda i: (i, 0)),
            pl.BlockSpec((H, H), lambda i: (0, 0)),
            pl.BlockSpec((H, H), lambda i: (0, 0)),
            pl.BlockSpec((1, H), lambda i: (0, 0)),
        ],
        out_specs=pl.BlockSpec((BN, H), lambda i: (i, 0)),
        out_shape=jax.ShapeDtypeStruct((N, H), jnp.float32),
    )(h, agg, wuh, wua, b_u.reshape(1, H))


def _readout_body(h_ref, w1_ref, b1_ref, w2_ref, b2_ref, ms_ref, o_ref):
    g = pl.program_id(0)
    y = jnp.sum(h_ref[...], axis=0, keepdims=True)
    t = _mm(y, w1_ref[...]) + b1_ref[...]
    t = t * jax.nn.sigmoid(t)
    out = _mm(t, w2_ref[...]) + b2_ref[...]
    o_ref[pl.ds(g, 1), :] = ms_ref[0, 1] * out + ms_ref[0, 0]


def _readout(h, W1, b1, W2, b2, mean, std):
    per_g = N // NG
    ms = jnp.stack([mean, std]).reshape(1, 2)
    return pl.pallas_call(
        _readout_body,
        grid=(NG,),
        in_specs=[
            pl.BlockSpec((per_g, H), lambda g: (g, 0)),
            pl.BlockSpec((H, H), lambda g: (0, 0)),
            pl.BlockSpec((1, H), lambda g: (0, 0)),
            pl.BlockSpec((H, 1), lambda g: (0, 0)),
            pl.BlockSpec((1, 1), lambda g: (0, 0)),
            pl.BlockSpec((1, 2), lambda g: (0, 0)),
        ],
        out_specs=pl.BlockSpec((NG, 1), lambda g: (0, 0)),
        out_shape=jax.ShapeDtypeStruct((NG, 1), jnp.float32),
    )(h, W1, b1.reshape(1, H), W2, b2.reshape(1, 1), ms)


def kernel(h, x, params, edge_index, n_node):
    layers = params['layers']
    src1d = edge_index[0]
    dst1d = edge_index[1]
    dst3d = dst1d.reshape(NSCHUNK, SIROWS, 128)
    x16 = jnp.pad(x, ((0, 0), (0, 13)))
    mu_pad = jnp.pad(jnp.linspace(0.0, 8.0, NRBF), (0, H - NRBF)).reshape(1, H)
    zeros = jnp.zeros((SCHUNK, H), jnp.float32)

    h0, ts, td = _prep_nodes(h, params['W_in'], params['b_in'],
                             layers[0]['W_e'][:H], layers[0]['W_e'][H:2 * H])
    li0, li1 = _prep_li(dst3d)
    xs, xd = _gather_x(x16, x16, src1d, dst1d)
    d = _prep_d(xs, xd)

    hcur = h0
    for l, lp in enumerate(layers):
        gs, gd = _gather_h(ts, td, src1d, dst1d)
        wer_pad = jnp.pad(lp['W_e'][2 * H:], ((0, H - NRBF), (0, 0)))
        contrib = _edge(gs, gd, d, mu_pad, wer_pad, lp['b_e'], lp['w_a'])
        agg = _scatter(contrib, li0, li1, zeros)
        if l + 1 < len(layers):
            nxt = layers[l + 1]
            hcur, ts, td = _node(hcur, agg, lp['W_u'][:H], lp['W_u'][H:],
                                 lp['b_u'], nxt['W_e'][:H], nxt['W_e'][H:2 * H])
        else:
            hcur = _node_last(hcur, agg, lp['W_u'][:H], lp['W_u'][H:],
                              lp['b_u'])

    return _readout(hcur, params['W1'], params['b1'], params['W2'],
                    params['b2'], params['mean'], params['std'])
